# Initial kernel scaffold; baseline (speedup 1.0000x reference)
import jax
import jax.numpy as jnp
from jax import lax
from jax.experimental import pallas as pl
from jax.experimental.pallas import tpu as pltpu
from jax.experimental.pallas import tpu_sc as plsc

B = 1024
S = 200
H = 128
N = B * S
LANES = 16
NUM_WORKERS = 32
PER_W = N // NUM_WORKERS
CHUNK = 128
NCHUNK = PER_W // CHUNK
NSEG = 3


def _sc_body(ids_hbm, sids_hbm, tok_hbm, seg_hbm, pos_hbm, out_hbm,
             segpos_v, seg_v, idx_v, sidx_v, rows_v, sem):
    info = plsc.get_sparse_core_info()
    nc = info.num_cores
    wid = lax.axis_index("s") * nc + lax.axis_index("c")

    pltpu.sync_copy(seg_hbm, seg_v)
    for g in range(NSEG):
        pltpu.sync_copy(pos_hbm.at[pl.ds(0, S)], segpos_v.at[pl.ds(g * S, S)])
    for g in range(NSEG):
        svregs = [seg_v[g, pl.ds(16 * j, 16)] for j in range(H // 16)]

        def seg_add(p, _, g=g, svregs=svregs):
            for j in range(H // 16):
                plsc.addupdate(segpos_v.at[g * S + p, pl.ds(16 * j, 16)],
                               svregs[j])
            return 0

        lax.fori_loop(0, S, seg_add, 0)

    iota = lax.iota(jnp.int32, LANES)

    def chunk_body(c, _):
        base = wid * PER_W + c * CHUNK
        pltpu.sync_copy(ids_hbm.at[pl.ds(base, CHUNK)], idx_v)
        pltpu.sync_copy(sids_hbm.at[pl.ds(base, CHUNK)], sidx_v)
        pltpu.async_copy(tok_hbm.at[idx_v], rows_v, sem).wait()

        for k in range(CHUNK // LANES):
            rvec = k * LANES + iota
            pvec = lax.rem(base + rvec, S)
            svec = sidx_v[pl.ds(k * LANES, LANES)]
            spvec = svec * S + pvec

            def col_body(h, cvec, rvec=rvec, spvec=spvec):
                g = plsc.load_gather(segpos_v, [spvec, cvec])
                plsc.addupdate_scatter(rows_v, [rvec, cvec], g)
                return cvec + 1

            lax.fori_loop(0, H, col_body, jnp.zeros((LANES,), jnp.int32),
                          unroll=4)

        pltpu.sync_copy(rows_v, out_hbm.at[pl.ds(base, CHUNK)])
        return 0

    lax.fori_loop(0, NCHUNK, chunk_body, 0)


@jax.jit
def kernel(input_ids, segment_ids, token_table, segment_table, pos_table):
    mesh = plsc.VectorSubcoreMesh(core_axis_name="c", subcore_axis_name="s")
    kfn = pl.kernel(
        _sc_body,
        out_type=jax.ShapeDtypeStruct((N, H), jnp.float32),
        mesh=mesh,
        scratch_types=[
            pltpu.VMEM((NSEG * S, H), jnp.float32),
            pltpu.VMEM((NSEG, H), jnp.float32),
            pltpu.VMEM((CHUNK,), jnp.int32),
            pltpu.VMEM((CHUNK,), jnp.int32),
            pltpu.VMEM((CHUNK, H), jnp.float32),
            pltpu.SemaphoreType.DMA,
        ],
    )
    out = kfn(input_ids.reshape(N).astype(jnp.int32),
              segment_ids.reshape(N).astype(jnp.int32),
              token_table, segment_table, pos_table)
    return out.reshape(B, S, H)


# speedup vs baseline: 5.1374x; 5.1374x over previous
import jax
import jax.numpy as jnp
from jax import lax
from jax.experimental import pallas as pl
from jax.experimental.pallas import tpu as pltpu
from jax.experimental.pallas import tpu_sc as plsc

B = 1024
S = 200
H = 128
N = B * S
LANES = 16
NUM_WORKERS = 32
PER_W = N // NUM_WORKERS
CHUNK = 128
NCHUNK = PER_W // CHUNK
NSEG = 3
SP = NSEG * S


def _sc_body(ids_hbm, sids_hbm, tok_hbm, seg_hbm, pos_hbm,
             out_hbm, segpos_hbm,
             segpos_v, seg_v, idx_v, sidx_v, spidx_v, rows_v, rows2_v,
             sem, sem2):
    info = plsc.get_sparse_core_info()
    nc = info.num_cores
    wid = lax.axis_index("s") * nc + lax.axis_index("c")

    pltpu.sync_copy(seg_hbm, seg_v)
    for g in range(NSEG):
        pltpu.sync_copy(pos_hbm.at[pl.ds(0, S)], segpos_v.at[pl.ds(g * S, S)])
    for g in range(NSEG):
        svregs = [seg_v[g, pl.ds(16 * j, 16)] for j in range(H // 16)]

        def seg_add(p, _, g=g, svregs=svregs):
            for j in range(H // 16):
                plsc.addupdate(segpos_v.at[g * S + p, pl.ds(16 * j, 16)],
                               svregs[j])
            return 0

        lax.fori_loop(0, S, seg_add, 0)
    pltpu.sync_copy(segpos_v, segpos_hbm.at[pl.ds(wid * SP, SP)])

    iota = lax.iota(jnp.int32, LANES)

    def chunk_body(c, _):
        base = wid * PER_W + c * CHUNK
        pltpu.sync_copy(ids_hbm.at[pl.ds(base, CHUNK)], idx_v)
        pltpu.sync_copy(sids_hbm.at[pl.ds(base, CHUNK)], sidx_v)
        tok_cp = pltpu.async_copy(tok_hbm.at[idx_v], rows_v, sem)

        for k in range(CHUNK // LANES):
            pvec = lax.rem(base + k * LANES + iota, S)
            svec = sidx_v[pl.ds(k * LANES, LANES)]
            spidx_v[pl.ds(k * LANES, LANES)] = wid * SP + svec * S + pvec

        pltpu.async_copy(segpos_hbm.at[spidx_v], rows2_v, sem2).wait()
        tok_cp.wait()

        def add_row(r, _):
            for j in range(H // LANES):
                plsc.addupdate(rows_v.at[r, pl.ds(LANES * j, LANES)],
                               rows2_v[r, pl.ds(LANES * j, LANES)])
            return 0

        lax.fori_loop(0, CHUNK, add_row, 0, unroll=2)

        pltpu.sync_copy(rows_v, out_hbm.at[pl.ds(base, CHUNK)])
        return 0

    lax.fori_loop(0, NCHUNK, chunk_body, 0)


@jax.jit
def kernel(input_ids, segment_ids, token_table, segment_table, pos_table):
    mesh = plsc.VectorSubcoreMesh(core_axis_name="c", subcore_axis_name="s")
    kfn = pl.kernel(
        _sc_body,
        out_type=(
            jax.ShapeDtypeStruct((N, H), jnp.float32),
            jax.ShapeDtypeStruct((NUM_WORKERS * SP, H), jnp.float32),
        ),
        mesh=mesh,
        scratch_types=[
            pltpu.VMEM((SP, H), jnp.float32),
            pltpu.VMEM((NSEG, H), jnp.float32),
            pltpu.VMEM((CHUNK,), jnp.int32),
            pltpu.VMEM((CHUNK,), jnp.int32),
            pltpu.VMEM((CHUNK,), jnp.int32),
            pltpu.VMEM((CHUNK, H), jnp.float32),
            pltpu.VMEM((CHUNK, H), jnp.float32),
            pltpu.SemaphoreType.DMA,
            pltpu.SemaphoreType.DMA,
        ],
    )
    out, _ = kfn(input_ids.reshape(N).astype(jnp.int32),
                 segment_ids.reshape(N).astype(jnp.int32),
                 token_table, segment_table, pos_table)
    return out.reshape(B, S, H)


# speedup vs baseline: 7.7695x; 1.5123x over previous
import jax
import jax.numpy as jnp
from jax import lax
from jax.experimental import pallas as pl
from jax.experimental.pallas import tpu as pltpu
from jax.experimental.pallas import tpu_sc as plsc

B = 1024
S = 200
H = 128
N = B * S
LANES = 16
NUM_WORKERS = 32
PER_W = N // NUM_WORKERS
CHUNK = 128
NCHUNK = PER_W // CHUNK
NSEG = 3
SP = NSEG * S


def _segpos_tc_body(seg_ref, pos_ref, out_ref):
    for g in range(NSEG):
        out_ref[g * S:(g + 1) * S, :] = pos_ref[...] + seg_ref[g:g + 1, :]


def _build_segpos(segment_table, pos_table):
    return pl.pallas_call(
        _segpos_tc_body,
        out_shape=jax.ShapeDtypeStruct((SP, H), jnp.float32),
    )(segment_table, pos_table[:S])


def _sc_body(ids_hbm, sids_hbm, tok_hbm, segpos_hbm, out_hbm,
             idx_all, spidx_all, rows_a, rows2_a, rows_b, rows2_b,
             gsem_a, gsem_b, osem_a, osem_b):
    info = plsc.get_sparse_core_info()
    nc = info.num_cores
    wid = lax.axis_index("s") * nc + lax.axis_index("c")
    wbase = wid * PER_W
    iota = lax.iota(jnp.int32, LANES)

    rows = (rows_a, rows_b)
    rows2 = (rows2_a, rows2_b)
    gsem = (gsem_a, gsem_b)
    osem = (osem_a, osem_b)

    pltpu.sync_copy(ids_hbm.at[pl.ds(wbase, PER_W)], idx_all)
    pltpu.sync_copy(sids_hbm.at[pl.ds(wbase, PER_W)], spidx_all)

    def spidx_body(i, _):
        sv = spidx_all[pl.ds(i * LANES, LANES)]
        pv = lax.rem(wbase + i * LANES + iota, S)
        spidx_all[pl.ds(i * LANES, LANES)] = sv * S + pv
        return 0

    lax.fori_loop(0, PER_W // LANES, spidx_body, 0, unroll=4)

    def fire(ch, slot):
        pltpu.async_copy(tok_hbm.at[idx_all.at[pl.ds(ch * CHUNK, CHUNK)]],
                         rows[slot], gsem[slot])
        pltpu.async_copy(segpos_hbm.at[spidx_all.at[pl.ds(ch * CHUNK, CHUNK)]],
                         rows2[slot], gsem[slot])

    def wait_gathers(slot):
        for _ in range(2):
            pltpu.make_async_copy(tok_hbm.at[pl.ds(0, CHUNK)],
                                  rows[slot], gsem[slot]).wait()

    def add_pass(slot):
        def add_row(r, _):
            for j in range(H // LANES):
                plsc.addupdate(rows[slot].at[r, pl.ds(LANES * j, LANES)],
                               rows2[slot][r, pl.ds(LANES * j, LANES)])
            return 0

        lax.fori_loop(0, CHUNK, add_row, 0, unroll=2)

    def fire_out(ch, slot):
        pltpu.async_copy(rows[slot],
                         out_hbm.at[pl.ds(wbase + ch * CHUNK, CHUNK)],
                         osem[slot])

    def wait_out(slot):
        pltpu.make_async_copy(rows[slot], out_hbm.at[pl.ds(0, CHUNK)],
                              osem[slot]).wait()

    fire(0, 0)
    fire(1, 1)
    wait_gathers(0)
    add_pass(0)
    fire_out(0, 0)

    def pair_body(c2, _):
        ch = 2 * c2 + 1
        wait_out(0)
        fire(ch + 1, 0)
        wait_gathers(1)
        add_pass(1)
        fire_out(ch, 1)
        wait_out(1)
        fire(ch + 2, 1)
        wait_gathers(0)
        add_pass(0)
        fire_out(ch + 1, 0)
        return 0

    lax.fori_loop(0, (NCHUNK - 2) // 2, pair_body, 0)

    wait_out(0)
    wait_gathers(1)
    add_pass(1)
    fire_out(NCHUNK - 1, 1)
    wait_out(1)


@jax.jit
def kernel(input_ids, segment_ids, token_table, segment_table, pos_table):
    segpos = _build_segpos(segment_table, pos_table)
    mesh = plsc.VectorSubcoreMesh(core_axis_name="c", subcore_axis_name="s")
    kfn = pl.kernel(
        _sc_body,
        out_type=jax.ShapeDtypeStruct((N, H), jnp.float32),
        mesh=mesh,
        scratch_types=[
            pltpu.VMEM((PER_W,), jnp.int32),
            pltpu.VMEM((PER_W,), jnp.int32),
            pltpu.VMEM((CHUNK, H), jnp.float32),
            pltpu.VMEM((CHUNK, H), jnp.float32),
            pltpu.VMEM((CHUNK, H), jnp.float32),
            pltpu.VMEM((CHUNK, H), jnp.float32),
            pltpu.SemaphoreType.DMA,
            pltpu.SemaphoreType.DMA,
            pltpu.SemaphoreType.DMA,
            pltpu.SemaphoreType.DMA,
        ],
    )
    out = kfn(input_ids.reshape(N).astype(jnp.int32),
              segment_ids.reshape(N).astype(jnp.int32),
              token_table, segpos)
    return out.reshape(B, S, H)


# speedup vs baseline: 7.7721x; 1.0003x over previous
import jax
import jax.numpy as jnp
from jax import lax
from jax.experimental import pallas as pl
from jax.experimental.pallas import tpu as pltpu
from jax.experimental.pallas import tpu_sc as plsc

B = 1024
S = 200
H = 128
N = B * S
LANES = 16
NUM_WORKERS = 32
PER_W = N // NUM_WORKERS
CHUNK = 128
NCHUNK = PER_W // CHUNK
NSEG = 3
SP = NSEG * S
NBUF = 3


def _segpos_tc_body(seg_ref, pos_ref, out_ref):
    for g in range(NSEG):
        out_ref[g * S:(g + 1) * S, :] = pos_ref[...] + seg_ref[g:g + 1, :]


def _build_segpos(segment_table, pos_table):
    return pl.pallas_call(
        _segpos_tc_body,
        out_shape=jax.ShapeDtypeStruct((SP, H), jnp.float32),
    )(segment_table, pos_table[:S])


def _sc_body(ids_hbm, sids_hbm, tok_hbm, segpos_hbm, out_hbm,
             idx_all, spidx_all, rows_0, rows_1, rows_2,
             tsem_0, tsem_1, tsem_2, asem_0, asem_1, asem_2,
             osem_0, osem_1, osem_2):
    info = plsc.get_sparse_core_info()
    nc = info.num_cores
    wid = lax.axis_index("s") * nc + lax.axis_index("c")
    wbase = wid * PER_W
    iota = lax.iota(jnp.int32, LANES)

    rows = (rows_0, rows_1, rows_2)
    tsem = (tsem_0, tsem_1, tsem_2)
    asem = (asem_0, asem_1, asem_2)
    osem = (osem_0, osem_1, osem_2)

    pltpu.sync_copy(ids_hbm.at[pl.ds(wbase, PER_W)], idx_all)
    pltpu.sync_copy(sids_hbm.at[pl.ds(wbase, PER_W)], spidx_all)

    def spidx_body(i, _):
        sv = spidx_all[pl.ds(i * LANES, LANES)]
        pv = lax.rem(wbase + i * LANES + iota, S)
        spidx_all[pl.ds(i * LANES, LANES)] = sv * S + pv
        return 0

    lax.fori_loop(0, PER_W // LANES, spidx_body, 0, unroll=4)

    def fire_tok(ch, s):
        pltpu.async_copy(tok_hbm.at[idx_all.at[pl.ds(ch * CHUNK, CHUNK)]],
                         rows[s], tsem[s])

    def wait_tok(s):
        pltpu.make_async_copy(tok_hbm.at[pl.ds(0, CHUNK)],
                              rows[s], tsem[s]).wait()

    def fire_spadd(ch, s):
        pltpu.async_copy(segpos_hbm.at[spidx_all.at[pl.ds(ch * CHUNK, CHUNK)]],
                         rows[s], asem[s], add=True)

    def wait_spadd(s):
        pltpu.make_async_copy(segpos_hbm.at[pl.ds(0, CHUNK)],
                              rows[s], asem[s]).wait()

    def fire_out(ch, s):
        pltpu.async_copy(rows[s],
                         out_hbm.at[pl.ds(wbase + ch * CHUNK, CHUNK)],
                         osem[s])

    def wait_out(s):
        pltpu.make_async_copy(rows[s], out_hbm.at[pl.ds(0, CHUNK)],
                              osem[s]).wait()

    def steady(ch, s, sp, sn):
        wait_tok(s)
        fire_spadd(ch, s)
        wait_spadd(sp)
        fire_out(ch - 1, sp)
        wait_out(sn)
        fire_tok(ch + 1, sn)

    fire_tok(0, 0)
    wait_tok(0)
    fire_spadd(0, 0)
    fire_tok(1, 1)
    wait_tok(1)
    fire_spadd(1, 1)
    wait_spadd(0)
    fire_out(0, 0)
    fire_tok(2, 2)

    def tri_body(c3, _):
        ch = 3 * c3 + 2
        steady(ch, 2, 1, 0)
        steady(ch + 1, 0, 2, 1)
        steady(ch + 2, 1, 0, 2)
        return 0

    lax.fori_loop(0, (NCHUNK - 5) // 3, tri_body, 0)

    steady(47, 2, 1, 0)
    steady(48, 0, 2, 1)
    wait_tok(1)
    fire_spadd(49, 1)
    wait_spadd(0)
    fire_out(48, 0)
    wait_spadd(1)
    fire_out(49, 1)
    wait_out(2)
    wait_out(0)
    wait_out(1)


@jax.jit
def kernel(input_ids, segment_ids, token_table, segment_table, pos_table):
    segpos = _build_segpos(segment_table, pos_table)
    mesh = plsc.VectorSubcoreMesh(core_axis_name="c", subcore_axis_name="s")
    kfn = pl.kernel(
        _sc_body,
        out_type=jax.ShapeDtypeStruct((N, H), jnp.float32),
        mesh=mesh,
        scratch_types=[
            pltpu.VMEM((PER_W,), jnp.int32),
            pltpu.VMEM((PER_W,), jnp.int32),
            pltpu.VMEM((CHUNK, H), jnp.float32),
            pltpu.VMEM((CHUNK, H), jnp.float32),
            pltpu.VMEM((CHUNK, H), jnp.float32),
            pltpu.SemaphoreType.DMA,
            pltpu.SemaphoreType.DMA,
            pltpu.SemaphoreType.DMA,
            pltpu.SemaphoreType.DMA,
            pltpu.SemaphoreType.DMA,
            pltpu.SemaphoreType.DMA,
            pltpu.SemaphoreType.DMA,
            pltpu.SemaphoreType.DMA,
            pltpu.SemaphoreType.DMA,
        ],
    )
    out = kfn(input_ids.reshape(N).astype(jnp.int32),
              segment_ids.reshape(N).astype(jnp.int32),
              token_table, segpos)
    return out.reshape(B, S, H)
